# baseline (speedup 1.0000x reference)
import functools

import jax
import jax.numpy as jnp
from jax import lax
from jax.experimental import pallas as pl
from jax.experimental.pallas import tpu as pltpu, tpu_sc as plsc

ROWS = 4096
COLS = 26
VOCAB = 1000
PLANE = COLS * VOCAB
TOTAL = ROWS * PLANE
NUM_WORKERS = 32
PLANES_PER_WORKER = ROWS // NUM_WORKERS
POS_PER_WORKER = PLANES_PER_WORKER * COLS
L = 16
K = 128
CHUNKS = POS_PER_WORKER // K


def _scatter_body(x_hbm, out_ref, xs_v, idx_v, ones_v, sem):
    wid = lax.axis_index("c") * 16 + lax.axis_index("s")
    base = wid * PLANES_PER_WORKER

    pltpu.sync_copy(x_hbm.at[pl.ds(base, PLANES_PER_WORKER)], xs_v)

    iota = lax.iota(jnp.int32, L)
    for m in range(K // L):
        ones_v[pl.ds(m * L, L)] = jnp.full((L,), 1.0, jnp.float32)

    def fill_idx(c, carry):
        for m in range(K // L):
            r = c * K + m * L + iota
            poff = r // COLS
            j = r - poff * COLS
            cols = plsc.load_gather(xs_v, [poff, j])
            idx_v[c, pl.ds(m * L, L)] = (
                (base + poff) * PLANE + j * VOCAB + cols)
        return carry

    lax.fori_loop(0, CHUNKS, fill_idx, 0)

    def fire(c, carry):
        pltpu.async_copy(ones_v, out_ref.at[idx_v.at[c]], sem)
        return carry

    lax.fori_loop(0, CHUNKS, fire, 0)

    def drain(c, carry):
        pltpu.make_async_copy(ones_v, out_ref.at[idx_v.at[0]], sem).wait()
        return carry

    lax.fori_loop(0, CHUNKS, drain, 0)


_scatter = functools.partial(
    pl.kernel,
    mesh=plsc.VectorSubcoreMesh(core_axis_name="c", subcore_axis_name="s"),
    compiler_params=pltpu.CompilerParams(
        use_tc_tiling_on_sc=False, needs_layout_passes=False),
    scratch_types=[
        pltpu.VMEM((PLANES_PER_WORKER, COLS), jnp.int32),
        pltpu.VMEM((CHUNKS, K), jnp.int32),
        pltpu.VMEM((K,), jnp.float32),
        pltpu.SemaphoreType.DMA,
    ],
)(_scatter_body)


def kernel(x):
    out = jax.new_ref(jnp.zeros((TOTAL,), jnp.float32))
    _scatter(x, out)
    return out[...].reshape(ROWS, COLS, VOCAB)


# speedup vs baseline: 1.0123x; 1.0123x over previous
import functools

import jax
import jax.numpy as jnp
from jax import lax
from jax.experimental import pallas as pl
from jax.experimental.pallas import tpu as pltpu, tpu_sc as plsc

ROWS = 4096
COLS = 26
VOCAB = 1000
PLANE = COLS * VOCAB
TOTAL = ROWS * PLANE
NUM_WORKERS = 32
PLANES_PER_WORKER = ROWS // NUM_WORKERS
POS_PER_WORKER = PLANES_PER_WORKER * COLS
L = 16
K = 64
CHUNKS = POS_PER_WORKER // K
QUARTERS = 4
QPLANES = PLANES_PER_WORKER // QUARTERS
BLKF = QPLANES * PLANE
QCHUNKS = CHUNKS // QUARTERS


def _body(x_hbm, zeros_hbm, out_hbm, zblock, xs_v, idx_v, ones_v,
          bsem0, bsem1, ssem):
    sid = lax.axis_index("s")
    wid = lax.axis_index("c") * 16 + sid
    base_plane = wid * PLANES_PER_WORKER
    base_f = base_plane * PLANE
    bsems = (bsem0, bsem1)

    @pl.when(sid == 0)
    def _():
        pltpu.sync_copy(zeros_hbm, zblock)

    plsc.subcore_barrier()

    def fire_bulk(q):
        pltpu.async_copy(
            zblock, out_hbm.at[pl.ds(base_f + q * BLKF, BLKF)],
            bsems[q % 2])

    fire_bulk(0)

    pltpu.sync_copy(x_hbm.at[pl.ds(base_plane, PLANES_PER_WORKER)], xs_v)
    iota = lax.iota(jnp.int32, L)
    for m in range(K // L):
        ones_v[pl.ds(m * L, L)] = jnp.full((L,), 1.0, jnp.float32)

    def fill_idx(c, carry):
        for m in range(K // L):
            r = c * K + m * L + iota
            poff = r // COLS
            j = r - poff * COLS
            cols = plsc.load_gather(xs_v, [poff, j])
            idx_v[c, pl.ds(m * L, L)] = (
                (base_plane + poff) * PLANE + j * VOCAB + cols)
        return carry

    lax.fori_loop(0, CHUNKS, fill_idx, 0)

    for q in range(QUARTERS):
        if q + 1 < QUARTERS:
            fire_bulk(q + 1)
        pltpu.make_async_copy(
            zeros_hbm, out_hbm.at[pl.ds(base_f + q * BLKF, BLKF)],
            bsems[q % 2]).wait()

        def fire_scatter(c, carry):
            pltpu.async_copy(ones_v, out_hbm.at[idx_v.at[c]], ssem)
            return carry

        lax.fori_loop(q * QCHUNKS, (q + 1) * QCHUNKS, fire_scatter, 0)

    def drain(c, carry):
        pltpu.make_async_copy(ones_v, out_hbm.at[idx_v.at[0]], ssem).wait()
        return carry

    lax.fori_loop(0, CHUNKS, drain, 0)


_onehot_sc = functools.partial(
    pl.kernel,
    out_type=jax.ShapeDtypeStruct((TOTAL,), jnp.float32),
    mesh=plsc.VectorSubcoreMesh(core_axis_name="c", subcore_axis_name="s"),
    compiler_params=pltpu.CompilerParams(
        use_tc_tiling_on_sc=False, needs_layout_passes=False),
    scratch_types=[
        pltpu.VMEM_SHARED((BLKF,), jnp.float32),
        pltpu.VMEM((PLANES_PER_WORKER, COLS), jnp.int32),
        pltpu.VMEM((CHUNKS, K), jnp.int32),
        pltpu.VMEM((K,), jnp.float32),
        pltpu.SemaphoreType.DMA,
        pltpu.SemaphoreType.DMA,
        pltpu.SemaphoreType.DMA,
    ],
)(_body)


def kernel(x):
    zeros = jnp.zeros((BLKF,), jnp.float32)
    return _onehot_sc(x, zeros).reshape(ROWS, COLS, VOCAB)
